# baseline (speedup 1.0000x reference)
import functools

import jax
import jax.numpy as jnp
from jax import lax
from jax.experimental import pallas as pl
from jax.experimental.pallas import tpu as pltpu
from jax.experimental.pallas import tpu_sc as plsc

B = 16384
D = 64
N_ROWS = 100000
IDX_CHUNK = 128


def _sc_geometry():
    try:
        info = plsc.get_sparse_core_info()
        return info.num_cores, info.num_subcores
    except Exception:
        return 2, 16


def _repack_body(t_ref, out_ref):
    x = t_ref[...]
    out_ref[:, 0:D] = x
    out_ref[:, D:2 * D] = x


def _repack(table):
    RB = 5000
    return pl.pallas_call(
        _repack_body,
        grid=(N_ROWS // RB,),
        in_specs=[pl.BlockSpec((RB, D), lambda i: (i, 0))],
        out_specs=pl.BlockSpec((RB, 2 * D), lambda i: (i, 0)),
        out_shape=jax.ShapeDtypeStruct((N_ROWS, 2 * D), jnp.float32),
    )(table)


@functools.cache
def _make_gather(NC, NS):
    NW = NC * NS
    bpw = B // NW
    nch = bpw // IDX_CHUNK
    mesh = plsc.VectorSubcoreMesh(core_axis_name="c", subcore_axis_name="s")

    @functools.partial(
        pl.kernel,
        out_type=jax.ShapeDtypeStruct((B, 2 * D), jnp.float32),
        mesh=mesh,
        scratch_types=[
            pltpu.VMEM((nch, IDX_CHUNK), jnp.int32),
            pltpu.VMEM((bpw, 2 * D), jnp.float32),
            pltpu.SemaphoreType.DMA,
        ],
    )
    def gather(ids_hbm, tab_hbm, out_hbm, idx, rows, sem):
        wid = lax.axis_index("s") * NC + lax.axis_index("c")
        base = wid * bpw
        pltpu.sync_copy(ids_hbm.at[wid], idx)
        copies = []
        for j in range(nch):
            dst = pl.ds(j * IDX_CHUNK, IDX_CHUNK)
            copies.append(pltpu.async_copy(tab_hbm.at[idx.at[j]], rows.at[dst], sem))
        for cp in copies:
            cp.wait()
        pltpu.sync_copy(rows, out_hbm.at[pl.ds(base, bpw)])

    return gather


def _mlp_body(gu_ref, gv_ref, w1_ref, b1_ref, w2_ref, b2_ref, w3_ref, b3_ref,
              w4_ref, b4_ref, out_ref):
    f32 = jnp.float32
    u = gu_ref[:, 0:D]
    v = gv_ref[:, 0:D]
    x = (jnp.dot(u, w1_ref[0:D, :], preferred_element_type=f32)
         + jnp.dot(v, w1_ref[D:2 * D, :], preferred_element_type=f32)
         + b1_ref[...])
    x = jnp.maximum(x, 0.0)
    x = jnp.maximum(jnp.dot(x, w2_ref[...], preferred_element_type=f32) + b2_ref[...], 0.0)
    x = jnp.maximum(jnp.dot(x, w3_ref[...], preferred_element_type=f32) + b3_ref[...], 0.0)
    logit = jnp.sum(x * w4_ref[...], axis=1, keepdims=True) + b4_ref[...]
    out_ref[...] = jax.nn.sigmoid(logit)


def kernel(user_ids, item_ids, user_emb, item_emb, W1, b1, W2, b2, W3, b3, W4, b4):
    NC, NS = _sc_geometry()
    NW = NC * NS
    nch = B // NW // IDX_CHUNK
    uids3 = user_ids.astype(jnp.int32).reshape(NW, nch, IDX_CHUNK)
    iids3 = item_ids.astype(jnp.int32).reshape(NW, nch, IDX_CHUNK)

    upk = _repack(user_emb)
    ipk = _repack(item_emb)
    gather = _make_gather(NC, NS)
    gu = gather(uids3, upk)
    gv = gather(iids3, ipk)

    BB = 2048
    full = lambda shape: pl.BlockSpec(shape, lambda i: (0, 0))
    out = pl.pallas_call(
        _mlp_body,
        grid=(B // BB,),
        in_specs=[
            pl.BlockSpec((BB, 2 * D), lambda i: (i, 0)),
            pl.BlockSpec((BB, 2 * D), lambda i: (i, 0)),
            full(W1.shape),
            full((1, 128)),
            full(W2.shape),
            full((1, 64)),
            full(W3.shape),
            full((1, 32)),
            full((1, 32)),
            full((1, 1)),
        ],
        out_specs=pl.BlockSpec((BB, 1), lambda i: (i, 0)),
        out_shape=jax.ShapeDtypeStruct((B, 1), jnp.float32),
    )(gu, gv, W1, b1.reshape(1, -1), W2, b2.reshape(1, -1), W3, b3.reshape(1, -1),
      W4.reshape(1, -1), b4.reshape(1, -1))
    return out


# speedup vs baseline: 1.7545x; 1.7545x over previous
import functools

import jax
import jax.numpy as jnp
from jax import lax
from jax.experimental import pallas as pl
from jax.experimental.pallas import tpu as pltpu
from jax.experimental.pallas import tpu_sc as plsc

B = 16384
D = 64
N_ROWS = 100000
P_ROWS = 100096
RB = 5888
IDX_CHUNK = 128


def _sc_geometry():
    try:
        info = plsc.get_sparse_core_info()
        return info.num_cores, info.num_subcores
    except Exception:
        return 2, 16


def _precompute_body(tabT_ref, w_ref, out_ref):
    out_ref[...] = jax.lax.dot_general(
        tabT_ref[...], w_ref[...], (((0,), (0,)), ((), ())),
        preferred_element_type=jnp.float32)


def _precompute(tabT, w):
    return pl.pallas_call(
        _precompute_body,
        grid=(P_ROWS // RB,),
        in_specs=[
            pl.BlockSpec((D, RB), lambda i: (0, i)),
            pl.BlockSpec((D, 2 * D), lambda i: (0, 0)),
        ],
        out_specs=pl.BlockSpec((RB, 2 * D), lambda i: (i, 0)),
        out_shape=jax.ShapeDtypeStruct((P_ROWS, 2 * D), jnp.float32),
    )(tabT, w)


@functools.cache
def _make_gather(NC, NS):
    NW = NC * NS
    bpw = B // NW
    nch = bpw // IDX_CHUNK
    mesh = plsc.VectorSubcoreMesh(core_axis_name="c", subcore_axis_name="s")

    @functools.partial(
        pl.kernel,
        out_type=jax.ShapeDtypeStruct((B, 2 * D), jnp.float32),
        mesh=mesh,
        scratch_types=[
            pltpu.VMEM((nch, IDX_CHUNK), jnp.int32),
            pltpu.VMEM((bpw, 2 * D), jnp.float32),
            pltpu.SemaphoreType.DMA,
        ],
    )
    def gather(ids_hbm, tab_hbm, out_hbm, idx, rows, sem):
        wid = lax.axis_index("s") * NC + lax.axis_index("c")
        base = wid * bpw
        pltpu.sync_copy(ids_hbm.at[wid], idx)
        copies = []
        for j in range(nch):
            dst = pl.ds(j * IDX_CHUNK, IDX_CHUNK)
            copies.append(pltpu.async_copy(tab_hbm.at[idx.at[j]], rows.at[dst], sem))
        for cp in copies:
            cp.wait()
        pltpu.sync_copy(rows, out_hbm.at[pl.ds(base, bpw)])

    return gather


def _mlp_body(gu_ref, gv_ref, b1_ref, w2_ref, b2_ref, w3_ref, b3_ref,
              w4_ref, b4_ref, out_ref):
    f32 = jnp.float32
    x = jnp.maximum(gu_ref[...] + gv_ref[...] + b1_ref[...], 0.0)
    x = jnp.maximum(jnp.dot(x, w2_ref[...], preferred_element_type=f32) + b2_ref[...], 0.0)
    x = jnp.maximum(jnp.dot(x, w3_ref[...], preferred_element_type=f32) + b3_ref[...], 0.0)
    logit = jnp.sum(x * w4_ref[...], axis=1, keepdims=True) + b4_ref[...]
    out_ref[...] = jax.nn.sigmoid(logit)


def kernel(user_ids, item_ids, user_emb, item_emb, W1, b1, W2, b2, W3, b3, W4, b4):
    NC, NS = _sc_geometry()
    NW = NC * NS
    nch = B // NW // IDX_CHUNK
    uids3 = user_ids.astype(jnp.int32).reshape(NW, nch, IDX_CHUNK)
    iids3 = item_ids.astype(jnp.int32).reshape(NW, nch, IDX_CHUNK)

    pu = _precompute(user_emb.T, W1[:D])
    pv = _precompute(item_emb.T, W1[D:])
    gather = _make_gather(NC, NS)
    gu = gather(uids3, pu)
    gv = gather(iids3, pv)

    BB = 2048
    full = lambda shape: pl.BlockSpec(shape, lambda i: (0, 0))
    out = pl.pallas_call(
        _mlp_body,
        grid=(B // BB,),
        in_specs=[
            pl.BlockSpec((BB, 2 * D), lambda i: (i, 0)),
            pl.BlockSpec((BB, 2 * D), lambda i: (i, 0)),
            full((1, 128)),
            full(W2.shape),
            full((1, 64)),
            full(W3.shape),
            full((1, 32)),
            full((1, 32)),
            full((1, 1)),
        ],
        out_specs=pl.BlockSpec((BB, 1), lambda i: (i, 0)),
        out_shape=jax.ShapeDtypeStruct((B, 1), jnp.float32),
    )(gu, gv, b1.reshape(1, -1), W2, b2.reshape(1, -1), W3, b3.reshape(1, -1),
      W4.reshape(1, -1), b4.reshape(1, -1))
    return out


# speedup vs baseline: 1.8746x; 1.0684x over previous
import functools

import jax
import jax.numpy as jnp
from jax import lax
from jax.experimental import pallas as pl
from jax.experimental.pallas import tpu as pltpu
from jax.experimental.pallas import tpu_sc as plsc

B = 16384
D = 64
N_ROWS = 100000
P_ROWS = 100096
RB = 5888
IDX_CHUNK = 128


def _sc_geometry():
    try:
        info = plsc.get_sparse_core_info()
        return info.num_cores, info.num_subcores
    except Exception:
        return 2, 16


def _precompute_body(tabT_ref, w_ref, b_ref, out_ref):
    out_ref[...] = jax.lax.dot_general(
        tabT_ref[...], w_ref[...], (((0,), (0,)), ((), ())),
        preferred_element_type=jnp.float32) + b_ref[...]


def _precompute(tabT, w, b):
    return pl.pallas_call(
        _precompute_body,
        grid=(P_ROWS // RB,),
        in_specs=[
            pl.BlockSpec((D, RB), lambda i: (0, i)),
            pl.BlockSpec((D, 2 * D), lambda i: (0, 0)),
            pl.BlockSpec((1, 2 * D), lambda i: (0, 0)),
        ],
        out_specs=pl.BlockSpec((RB, 2 * D), lambda i: (i, 0)),
        out_shape=jax.ShapeDtypeStruct((P_ROWS, 2 * D), jnp.float32),
    )(tabT, w, b)


@functools.cache
def _make_gather(NC, NS):
    NW = NC * NS
    bpw = B // NW
    nch = bpw // IDX_CHUNK
    mesh = plsc.VectorSubcoreMesh(core_axis_name="c", subcore_axis_name="s")

    @functools.partial(
        pl.kernel,
        out_type=jax.ShapeDtypeStruct((B, 2 * D), jnp.float32),
        mesh=mesh,
        scratch_types=[
            pltpu.VMEM((bpw,), jnp.int32),
            pltpu.VMEM((bpw, 2 * D), jnp.float32),
            pltpu.SemaphoreType.DMA,
        ],
    )
    def gather(ids_hbm, tab_hbm, out_hbm, idx, rows, sem):
        wid = lax.axis_index("s") * NC + lax.axis_index("c")
        base = wid * bpw
        pltpu.sync_copy(ids_hbm.at[pl.ds(base, bpw)], idx)
        copies = []
        for j in range(nch):
            sl = pl.ds(j * IDX_CHUNK, IDX_CHUNK)
            copies.append(pltpu.async_copy(tab_hbm.at[idx.at[sl]], rows.at[sl], sem))
        for cp in copies:
            cp.wait()
        pltpu.sync_copy(rows, out_hbm.at[pl.ds(base, bpw)])

    return gather


def _mlp_body(gu_ref, gv_ref, w2_ref, b2_ref, w3_ref, b3_ref,
              w4_ref, b4_ref, out_ref):
    f32 = jnp.float32
    x = jnp.maximum(gu_ref[...] + gv_ref[...], 0.0)
    x = jnp.maximum(jnp.dot(x, w2_ref[...], preferred_element_type=f32) + b2_ref[...], 0.0)
    x = jnp.maximum(jnp.dot(x, w3_ref[...], preferred_element_type=f32) + b3_ref[...], 0.0)
    logit = jax.lax.dot_general(
        w4_ref[...], x, (((1,), (1,)), ((), ())),
        preferred_element_type=f32) + b4_ref[...]
    out_ref[...] = jax.nn.sigmoid(logit)


def kernel(user_ids, item_ids, user_emb, item_emb, W1, b1, W2, b2, W3, b3, W4, b4):
    NC, NS = _sc_geometry()
    uids = user_ids.astype(jnp.int32)
    iids = item_ids.astype(jnp.int32)

    b1r = b1.reshape(1, -1)
    pu = _precompute(user_emb.T, W1[:D], b1r)
    pv = _precompute(item_emb.T, W1[D:], jnp.zeros_like(b1r))
    gather = _make_gather(NC, NS)
    gu = gather(uids, pu)
    gv = gather(iids, pv)

    BB = 2048
    full = lambda shape: pl.BlockSpec(shape, lambda i: (0, 0))
    outT = pl.pallas_call(
        _mlp_body,
        grid=(B // BB,),
        in_specs=[
            pl.BlockSpec((BB, 2 * D), lambda i: (i, 0)),
            pl.BlockSpec((BB, 2 * D), lambda i: (i, 0)),
            full(W2.shape),
            full((1, 64)),
            full(W3.shape),
            full((1, 32)),
            full((1, 32)),
            full((1, 1)),
        ],
        out_specs=pl.BlockSpec((1, BB), lambda i: (0, i)),
        out_shape=jax.ShapeDtypeStruct((1, B), jnp.float32),
    )(gu, gv, W2, b2.reshape(1, -1), W3, b3.reshape(1, -1),
      W4.reshape(1, -1), b4.reshape(1, -1))
    return outT.T


# speedup vs baseline: 1.9239x; 1.0263x over previous
import functools

import jax
import jax.numpy as jnp
from jax import lax
from jax.experimental import pallas as pl
from jax.experimental.pallas import tpu as pltpu
from jax.experimental.pallas import tpu_sc as plsc

B = 16384
D = 64
N_ROWS = 100000
HALF = 50048
RB = 2944
IDX_CHUNK = 128
HI_MASK = -65536


def _sc_geometry():
    try:
        info = plsc.get_sparse_core_info()
        return info.num_cores, info.num_subcores
    except Exception:
        return 2, 16


def _precompute_body(topT_ref, botT_ref, w_ref, b_ref, out_ref):
    f32 = jnp.float32
    dims = (((0,), (0,)), ((), ()))
    pa = jax.lax.dot_general(topT_ref[...], w_ref[...], dims,
                             preferred_element_type=f32) + b_ref[...]
    pb = jax.lax.dot_general(botT_ref[...], w_ref[...], dims,
                             preferred_element_type=f32) + b_ref[...]
    ia = jax.lax.bitcast_convert_type(pa, jnp.int32)
    ib = jax.lax.bitcast_convert_type(pb, jnp.int32)
    out_ref[...] = (ia & HI_MASK) | jax.lax.shift_right_logical(ib, 16)


def _precompute(tabT, w, b):
    return pl.pallas_call(
        _precompute_body,
        grid=(HALF // RB,),
        in_specs=[
            pl.BlockSpec((D, RB), lambda i: (0, i)),
            pl.BlockSpec((D, RB), lambda i: (0, i + HALF // RB)),
            pl.BlockSpec((D, 2 * D), lambda i: (0, 0)),
            pl.BlockSpec((1, 2 * D), lambda i: (0, 0)),
        ],
        out_specs=pl.BlockSpec((RB, 2 * D), lambda i: (i, 0)),
        out_shape=jax.ShapeDtypeStruct((HALF, 2 * D), jnp.int32),
    )(tabT, tabT, w, b)


@functools.cache
def _make_gather(NC, NS):
    NW = NC * NS
    bpw = B // NW
    nch = bpw // IDX_CHUNK
    mesh = plsc.VectorSubcoreMesh(core_axis_name="c", subcore_axis_name="s")

    @functools.partial(
        pl.kernel,
        out_type=jax.ShapeDtypeStruct((B, 2 * D), jnp.int32),
        mesh=mesh,
        scratch_types=[
            pltpu.VMEM((bpw,), jnp.int32),
            pltpu.VMEM((bpw, 2 * D), jnp.int32),
            pltpu.SemaphoreType.DMA,
        ],
    )
    def gather(ids_hbm, tab_hbm, out_hbm, idx, rows, sem):
        wid = lax.axis_index("s") * NC + lax.axis_index("c")
        base = wid * bpw
        pltpu.sync_copy(ids_hbm.at[pl.ds(base, bpw)], idx)
        copies = []
        for j in range(nch):
            sl = pl.ds(j * IDX_CHUNK, IDX_CHUNK)
            copies.append(pltpu.async_copy(tab_hbm.at[idx.at[sl]], rows.at[sl], sem))
        for cp in copies:
            cp.wait()
        pltpu.sync_copy(rows, out_hbm.at[pl.ds(base, bpw)])

    return gather


def _unpack(g, par_col):
    bits = jnp.where(par_col, jax.lax.shift_left(g, 16), g & HI_MASK)
    return jax.lax.bitcast_convert_type(bits, jnp.float32)


def _mlp_body(gu_ref, gv_ref, up_ref, vp_ref, w2_ref, b2_ref, w3_ref, b3_ref,
              w4_ref, b4_ref, out_ref):
    f32 = jnp.float32
    up_col = jnp.transpose(up_ref[...], (1, 0)) != 0
    vp_col = jnp.transpose(vp_ref[...], (1, 0)) != 0
    x = jnp.maximum(_unpack(gu_ref[...], up_col) + _unpack(gv_ref[...], vp_col), 0.0)
    x = jnp.maximum(jnp.dot(x, w2_ref[...], preferred_element_type=f32) + b2_ref[...], 0.0)
    x = jnp.maximum(jnp.dot(x, w3_ref[...], preferred_element_type=f32) + b3_ref[...], 0.0)
    logit = jax.lax.dot_general(
        w4_ref[...], x, (((1,), (1,)), ((), ())),
        preferred_element_type=f32) + b4_ref[...]
    out_ref[...] = jax.nn.sigmoid(logit)


def kernel(user_ids, item_ids, user_emb, item_emb, W1, b1, W2, b2, W3, b3, W4, b4):
    NC, NS = _sc_geometry()
    uids = user_ids.astype(jnp.int32)
    iids = item_ids.astype(jnp.int32)
    upar = (uids >= HALF)
    vpar = (iids >= HALF)
    uidsm = uids - jnp.where(upar, HALF, 0)
    iidsm = iids - jnp.where(vpar, HALF, 0)
    uparr = upar.astype(jnp.int32).reshape(1, B)
    vparr = vpar.astype(jnp.int32).reshape(1, B)

    b1r = b1.reshape(1, -1)
    pu = _precompute(user_emb.T, W1[:D], b1r)
    pv = _precompute(item_emb.T, W1[D:], jnp.zeros_like(b1r))
    gather = _make_gather(NC, NS)
    gu = gather(uidsm, pu)
    gv = gather(iidsm, pv)

    BB = 2048
    full = lambda shape: pl.BlockSpec(shape, lambda i: (0, 0))
    outT = pl.pallas_call(
        _mlp_body,
        grid=(B // BB,),
        in_specs=[
            pl.BlockSpec((BB, 2 * D), lambda i: (i, 0)),
            pl.BlockSpec((BB, 2 * D), lambda i: (i, 0)),
            pl.BlockSpec((1, BB), lambda i: (0, i)),
            pl.BlockSpec((1, BB), lambda i: (0, i)),
            full(W2.shape),
            full((1, 64)),
            full(W3.shape),
            full((1, 32)),
            full((1, 32)),
            full((1, 1)),
        ],
        out_specs=pl.BlockSpec((1, BB), lambda i: (0, i)),
        out_shape=jax.ShapeDtypeStruct((1, B), jnp.float32),
    )(gu, gv, uparr, vparr, W2, b2.reshape(1, -1), W3, b3.reshape(1, -1),
      W4.reshape(1, -1), b4.reshape(1, -1))
    return outT.T


# speedup vs baseline: 1.9555x; 1.0164x over previous
import functools

import jax
import jax.numpy as jnp
from jax import lax
from jax.experimental import pallas as pl
from jax.experimental.pallas import tpu as pltpu
from jax.experimental.pallas import tpu_sc as plsc

B = 16384
D = 64
N_ROWS = 100000
HALF = 50048
RB = 2944
IDX_CHUNK = 128
HI_MASK = -65536


def _sc_geometry():
    try:
        info = plsc.get_sparse_core_info()
        return info.num_cores, info.num_subcores
    except Exception:
        return 2, 16


def _precompute_body(topT_ref, botT_ref, w_ref, b_ref, out_ref):
    f32 = jnp.float32
    dims = (((0,), (0,)), ((), ()))
    pa = jax.lax.dot_general(topT_ref[...], w_ref[...], dims,
                             preferred_element_type=f32)
    pb = jax.lax.dot_general(botT_ref[...], w_ref[...], dims,
                             preferred_element_type=f32)
    if b_ref is not None:
        pa = pa + b_ref[...]
        pb = pb + b_ref[...]
    ia = jax.lax.bitcast_convert_type(pa, jnp.int32)
    ib = jax.lax.bitcast_convert_type(pb, jnp.int32)
    out_ref[...] = (ia & HI_MASK) | jax.lax.shift_right_logical(ib, 16)


def _precompute(tabT, w, b=None):
    bias_specs = [] if b is None else [pl.BlockSpec((1, 2 * D), lambda i: (0, 0))]
    body = ((lambda t, bt, w_, o: _precompute_body(t, bt, w_, None, o))
            if b is None else _precompute_body)
    args = (tabT, tabT, w) if b is None else (tabT, tabT, w, b)
    return pl.pallas_call(
        body,
        grid=(HALF // RB,),
        in_specs=[
            pl.BlockSpec((D, RB), lambda i: (0, i)),
            pl.BlockSpec((D, RB), lambda i: (0, i + HALF // RB)),
            pl.BlockSpec((D, 2 * D), lambda i: (0, 0)),
        ] + bias_specs,
        out_specs=pl.BlockSpec((RB, 2 * D), lambda i: (i, 0)),
        out_shape=jax.ShapeDtypeStruct((HALF, 2 * D), jnp.int32),
    )(*args)


@functools.cache
def _make_gather(NC, NS):
    NW = NC * NS
    bpw = B // NW
    nch = bpw // IDX_CHUNK
    mesh = plsc.VectorSubcoreMesh(core_axis_name="c", subcore_axis_name="s")

    @functools.partial(
        pl.kernel,
        out_type=jax.ShapeDtypeStruct((B, 2 * D), jnp.int32),
        mesh=mesh,
        scratch_types=[
            pltpu.VMEM((bpw,), jnp.int32),
            pltpu.VMEM((bpw, 2 * D), jnp.int32),
            pltpu.SemaphoreType.DMA,
        ],
    )
    def gather(ids_hbm, tab_hbm, out_hbm, idx, rows, sem):
        wid = lax.axis_index("s") * NC + lax.axis_index("c")
        base = wid * bpw
        pltpu.sync_copy(ids_hbm.at[pl.ds(base, bpw)], idx)
        for j in range(bpw // 16):
            sl16 = pl.ds(j * 16, 16)
            c = idx[sl16]
            idx[sl16] = c - jnp.where(c >= HALF, HALF, 0)
        copies = []
        for j in range(nch):
            sl = pl.ds(j * IDX_CHUNK, IDX_CHUNK)
            copies.append(pltpu.async_copy(tab_hbm.at[idx.at[sl]], rows.at[sl], sem))
        for cp in copies:
            cp.wait()
        pltpu.sync_copy(rows, out_hbm.at[pl.ds(base, bpw)])

    return gather


def _unpack(g, par_col):
    bits = jnp.where(par_col, jax.lax.shift_left(g, 16), g & HI_MASK)
    return jax.lax.bitcast_convert_type(bits, jnp.float32)


def _mlp_body(gu_ref, gv_ref, up_ref, vp_ref, w2_ref, b2_ref, w3_ref, b3_ref,
              w4_ref, b4_ref, out_ref):
    f32 = jnp.float32
    up_col = jnp.transpose(up_ref[0], (1, 0)) != 0
    vp_col = jnp.transpose(vp_ref[0], (1, 0)) != 0
    x = jnp.maximum(_unpack(gu_ref[...], up_col) + _unpack(gv_ref[...], vp_col), 0.0)
    x = jnp.maximum(jnp.dot(x, w2_ref[...], preferred_element_type=f32) + b2_ref[...], 0.0)
    x = jnp.maximum(jnp.dot(x, w3_ref[...], preferred_element_type=f32) + b3_ref[...], 0.0)
    logit = jax.lax.dot_general(
        w4_ref[...], x, (((1,), (1,)), ((), ())),
        preferred_element_type=f32) + b4_ref[...]
    out_ref[...] = jax.nn.sigmoid(logit)


def kernel(user_ids, item_ids, user_emb, item_emb, W1, b1, W2, b2, W3, b3, W4, b4):
    NC, NS = _sc_geometry()
    uids = user_ids.astype(jnp.int32)
    iids = item_ids.astype(jnp.int32)
    pars = (jnp.stack([uids, iids]) >= HALF).astype(jnp.int32).reshape(2, 1, B)

    pu = _precompute(user_emb.T, W1[:D], b1.reshape(1, -1))
    pv = _precompute(item_emb.T, W1[D:])
    gather = _make_gather(NC, NS)
    gu = gather(uids, pu)
    gv = gather(iids, pv)

    BB = 2048
    full = lambda shape: pl.BlockSpec(shape, lambda i: (0, 0))
    outT = pl.pallas_call(
        _mlp_body,
        grid=(B // BB,),
        in_specs=[
            pl.BlockSpec((BB, 2 * D), lambda i: (i, 0)),
            pl.BlockSpec((BB, 2 * D), lambda i: (i, 0)),
            pl.BlockSpec((1, 1, BB), lambda i: (0, 0, i)),
            pl.BlockSpec((1, 1, BB), lambda i: (1, 0, i)),
            full(W2.shape),
            full((1, 64)),
            full(W3.shape),
            full((1, 32)),
            full((1, 32)),
            full((1, 1)),
        ],
        out_specs=pl.BlockSpec((1, BB), lambda i: (0, i)),
        out_shape=jax.ShapeDtypeStruct((1, B), jnp.float32),
    )(gu, gv, pars, pars, W2, b2.reshape(1, -1), W3, b3.reshape(1, -1),
      W4.reshape(1, -1), b4.reshape(1, -1))
    return outT.T


# speedup vs baseline: 2.0795x; 1.0634x over previous
import functools

import jax
import jax.numpy as jnp
from jax import lax
from jax.experimental import pallas as pl
from jax.experimental.pallas import tpu as pltpu
from jax.experimental.pallas import tpu_sc as plsc

B = 16384
D = 64
N_ROWS = 100000
HALF = 50048
RB = 2944
IDX_CHUNK = 128
HI_MASK = -65536


def _sc_geometry():
    try:
        info = plsc.get_sparse_core_info()
        return info.num_cores, info.num_subcores
    except Exception:
        return 2, 16


def _precompute_body(topT_ref, botT_ref, w_ref, b_ref, out_ref):
    f32 = jnp.float32
    dims = (((0,), (0,)), ((), ()))
    w16 = w_ref[...].astype(jnp.bfloat16)
    pa = jax.lax.dot_general(topT_ref[...].astype(jnp.bfloat16), w16, dims,
                             preferred_element_type=f32)
    pb = jax.lax.dot_general(botT_ref[...].astype(jnp.bfloat16), w16, dims,
                             preferred_element_type=f32)
    if b_ref is not None:
        pa = pa + b_ref[...]
        pb = pb + b_ref[...]
    ia = jax.lax.bitcast_convert_type(pa, jnp.int32)
    ib = jax.lax.bitcast_convert_type(pb, jnp.int32)
    out_ref[...] = (ia & HI_MASK) | jax.lax.shift_right_logical(ib, 16)


def _precompute(tabT, w, b=None):
    bias_specs = [] if b is None else [pl.BlockSpec((1, 2 * D), lambda i: (0, 0))]
    body = ((lambda t, bt, w_, o: _precompute_body(t, bt, w_, None, o))
            if b is None else _precompute_body)
    args = (tabT, tabT, w) if b is None else (tabT, tabT, w, b)
    return pl.pallas_call(
        body,
        grid=(HALF // RB,),
        in_specs=[
            pl.BlockSpec((D, RB), lambda i: (0, i)),
            pl.BlockSpec((D, RB), lambda i: (0, i + HALF // RB)),
            pl.BlockSpec((D, 2 * D), lambda i: (0, 0)),
        ] + bias_specs,
        out_specs=pl.BlockSpec((RB, 2 * D), lambda i: (i, 0)),
        out_shape=jax.ShapeDtypeStruct((HALF, 2 * D), jnp.int32),
    )(*args)


@functools.cache
def _make_gather(NC, NS):
    NW = NC * NS
    bpw = B // NW
    nch = bpw // IDX_CHUNK
    mesh = plsc.VectorSubcoreMesh(core_axis_name="c", subcore_axis_name="s")

    @functools.partial(
        pl.kernel,
        out_type=jax.ShapeDtypeStruct((B, 2 * D), jnp.int32),
        mesh=mesh,
        scratch_types=[
            pltpu.VMEM((bpw,), jnp.int32),
            pltpu.VMEM((bpw, 2 * D), jnp.int32),
            pltpu.SemaphoreType.DMA,
        ],
    )
    def gather(ids_hbm, tab_hbm, out_hbm, idx, rows, sem):
        wid = lax.axis_index("s") * NC + lax.axis_index("c")
        base = wid * bpw
        pltpu.sync_copy(ids_hbm.at[pl.ds(base, bpw)], idx)
        for j in range(bpw // 16):
            sl16 = pl.ds(j * 16, 16)
            c = idx[sl16]
            idx[sl16] = c - jnp.where(c >= HALF, HALF, 0)
        copies = []
        for j in range(nch):
            sl = pl.ds(j * IDX_CHUNK, IDX_CHUNK)
            copies.append(pltpu.async_copy(tab_hbm.at[idx.at[sl]], rows.at[sl], sem))
        for cp in copies:
            cp.wait()
        pltpu.sync_copy(rows, out_hbm.at[pl.ds(base, bpw)])

    return gather


def _unpack(g, shift_col):
    bits = jax.lax.shift_left(g, shift_col) & HI_MASK
    return jax.lax.bitcast_convert_type(bits, jnp.float32)


def _mlp_body(gu_ref, gv_ref, up_ref, vp_ref, w2_ref, b2_ref, w3_ref, b3_ref,
              w4_ref, b4_ref, out_ref):
    f32 = jnp.float32
    up_col = jnp.transpose(up_ref[0], (1, 0))
    vp_col = jnp.transpose(vp_ref[0], (1, 0))
    x = jnp.maximum(_unpack(gu_ref[...], up_col) + _unpack(gv_ref[...], vp_col), 0.0)
    x = jnp.maximum(jnp.dot(x, w2_ref[...], preferred_element_type=f32) + b2_ref[...], 0.0)
    x = jnp.maximum(jnp.dot(x, w3_ref[...], preferred_element_type=f32) + b3_ref[...], 0.0)
    logit = jax.lax.dot_general(
        w4_ref[...], x, (((1,), (1,)), ((), ())),
        preferred_element_type=f32) + b4_ref[...]
    out_ref[...] = jax.nn.sigmoid(logit)


def kernel(user_ids, item_ids, user_emb, item_emb, W1, b1, W2, b2, W3, b3, W4, b4):
    NC, NS = _sc_geometry()
    uids = user_ids.astype(jnp.int32)
    iids = item_ids.astype(jnp.int32)
    pars = jnp.where(jnp.stack([uids, iids]) >= HALF, 16, 0).reshape(2, 1, B)

    pu = _precompute(user_emb.T, W1[:D], b1.reshape(1, -1))
    pv = _precompute(item_emb.T, W1[D:])
    gather = _make_gather(NC, NS)
    gu = gather(uids, pu)
    gv = gather(iids, pv)

    BB = 4096
    full = lambda shape: pl.BlockSpec(shape, lambda i: (0, 0))
    outT = pl.pallas_call(
        _mlp_body,
        grid=(B // BB,),
        in_specs=[
            pl.BlockSpec((BB, 2 * D), lambda i: (i, 0)),
            pl.BlockSpec((BB, 2 * D), lambda i: (i, 0)),
            pl.BlockSpec((1, 1, BB), lambda i: (0, 0, i)),
            pl.BlockSpec((1, 1, BB), lambda i: (1, 0, i)),
            full(W2.shape),
            full((1, 64)),
            full(W3.shape),
            full((1, 32)),
            full((1, 32)),
            full((1, 1)),
        ],
        out_specs=pl.BlockSpec((1, BB), lambda i: (0, i)),
        out_shape=jax.ShapeDtypeStruct((1, B), jnp.float32),
    )(gu, gv, pars, pars, W2, b2.reshape(1, -1), W3, b3.reshape(1, -1),
      W4.reshape(1, -1), b4.reshape(1, -1))
    return outT.T
